# baseline (speedup 1.0000x reference)
import functools

import jax
import jax.numpy as jnp
from jax import lax
from jax.experimental import pallas as pl
from jax.experimental.pallas import tpu as pltpu
from jax.experimental.pallas import tpu_sc as plsc

_NC = 2
_NS = 16
_NW = _NC * _NS


def _tc_combo(voc: int, d: int, tx: int, vp: int):
    def body(tok_ref, pos_ref, w_ref, b_ref, out_ref, tokl_scr):
        t = pl.program_id(0)

        @pl.when(t == 0)
        def _():
            tokl_scr[...] = jnp.dot(
                tok_ref[...], w_ref[...], preferred_element_type=jnp.float32
            )

        prow = (
            jnp.dot(
                pos_ref[pl.ds(t, 1), :],
                w_ref[...],
                preferred_element_type=jnp.float32,
            )
            + b_ref[...]
        )
        out_ref[...] = tokl_scr[...] + prow

    return pl.pallas_call(
        body,
        grid=(tx,),
        in_specs=[
            pl.BlockSpec((voc, d), lambda i: (0, 0)),
            pl.BlockSpec((tx, d), lambda i: (0, 0)),
            pl.BlockSpec((d, vp), lambda i: (0, 0)),
            pl.BlockSpec((1, vp), lambda i: (0, 0)),
        ],
        out_specs=pl.BlockSpec((voc, vp), lambda i: (i, 0)),
        out_shape=jax.ShapeDtypeStruct((tx * voc, vp), jnp.float32),
        scratch_shapes=[pltpu.VMEM((voc, vp), jnp.float32)],
    )


def _sc_head(bx: int, tx: int, v: int, vp: int, txp: int):
    n_per_w = bx // _NW
    mesh = plsc.VectorSubcoreMesh(core_axis_name="c", subcore_axis_name="s")

    @functools.partial(
        pl.kernel,
        mesh=mesh,
        out_type=jax.ShapeDtypeStruct((bx, tx, v), jnp.float32),
        scratch_types=[
            pltpu.VMEM((n_per_w * txp,), jnp.int32),
            pltpu.VMEM((txp, vp), jnp.float32),
            pltpu.VMEM((tx, v), jnp.float32),
            pltpu.SemaphoreType.DMA,
            pltpu.SemaphoreType.DMA,
        ],
    )
    def k(idx_hbm, combo_hbm, out_hbm, idx_v, bufa, buf3, gsem, wsem):
        wid = lax.axis_index("s") * _NC + lax.axis_index("c")
        sbase = wid * n_per_w
        pltpu.sync_copy(idx_hbm.at[pl.ds(sbase * txp, n_per_w * txp)], idx_v)

        def gather(s):
            return pltpu.make_async_copy(
                combo_hbm.at[idx_v.at[pl.ds(s * txp, txp)]], bufa, gsem
            )

        def write(s):
            return pltpu.make_async_copy(buf3, out_hbm.at[sbase + s], wsem)

        gather(0).start()

        def seq_body(s, carry):
            gather(s).wait()

            @pl.when(s > 0)
            def _():
                write(s - 1).wait()

            nk = v // 16

            @plsc.parallel_loop(0, tx, 1, unroll=2)
            def _(t):
                for kk in range(nk):
                    sl = pl.ds(kk * 16, 16)
                    buf3[t, sl] = bufa[t, sl]
                tl = pl.ds(v - 16, 16)
                buf3[t, tl] = bufa[t, tl]

            write(s).start()

            @pl.when(s < n_per_w - 1)
            def _():
                gather(s + 1).start()

            return carry

        lax.fori_loop(0, n_per_w, seq_body, 0)
        write(n_per_w - 1).wait()

    return k


def kernel(x, tok_table, pos_table, W, b):
    bx, tx = x.shape
    vocab, d = tok_table.shape
    v = W.shape[1]
    vp = 1024
    txp = 56

    w_pad = jnp.pad(W, ((0, 0), (0, vp - v)))
    b_pad = jnp.pad(b, (0, vp - v)).reshape(1, vp)
    combo = _tc_combo(vocab, d, tx, vp)(tok_table, pos_table, w_pad, b_pad)

    flat = x.astype(jnp.int32) + jnp.arange(tx, dtype=jnp.int32)[None, :] * vocab
    idx = jnp.pad(flat, ((0, 0), (0, txp - tx))).reshape(-1)
    return _sc_head(bx, tx, v, vp, txp)(idx, combo)


# speedup vs baseline: 2.4794x; 2.4794x over previous
import functools

import jax
import jax.numpy as jnp
from jax import lax
from jax.experimental import pallas as pl
from jax.experimental.pallas import tpu as pltpu
from jax.experimental.pallas import tpu_sc as plsc

_NC = 2
_NS = 16
_NW = _NC * _NS


def _sc_gather(n_tot: int, d: int, ch: int):
    n_per_w = n_tot // _NW
    nch = n_per_w // ch
    mesh = plsc.VectorSubcoreMesh(core_axis_name="c", subcore_axis_name="s")

    @functools.partial(
        pl.kernel,
        mesh=mesh,
        compiler_params=pltpu.CompilerParams(use_tc_tiling_on_sc=False),
        out_type=jax.ShapeDtypeStruct((n_tot, d), jnp.float32),
        scratch_types=[
            pltpu.VMEM((n_per_w,), jnp.int32),
            pltpu.VMEM((ch, d), jnp.float32),
            pltpu.VMEM((ch, d), jnp.float32),
            pltpu.SemaphoreType.DMA,
            pltpu.SemaphoreType.DMA,
            pltpu.SemaphoreType.DMA,
            pltpu.SemaphoreType.DMA,
        ],
    )
    def k(idx_hbm, table_hbm, out_hbm, idx_v, rows0, rows1, g0, g1, w0, w1):
        wid = lax.axis_index("s") * _NC + lax.axis_index("c")
        base = wid * n_per_w
        pltpu.sync_copy(idx_hbm.at[pl.ds(base, n_per_w)], idx_v)
        bufs = (rows0, rows1)
        gsem = (g0, g1)
        wsem = (w0, w1)

        def gather_start(c):
            idx_c = idx_v.at[pl.ds(c * ch, ch)]
            pltpu.async_copy(table_hbm.at[idx_c], bufs[c % 2], gsem[c % 2])

        def write_start(c):
            pltpu.async_copy(
                bufs[c % 2], out_hbm.at[pl.ds(base + c * ch, ch)], wsem[c % 2]
            )

        gather_start(0)
        for c in range(nch):
            pltpu.make_async_copy(
                table_hbm.at[idx_v.at[pl.ds(c * ch, ch)]], bufs[c % 2], gsem[c % 2]
            ).wait()
            write_start(c)
            if c + 1 < nch:
                if c >= 1:
                    pltpu.make_async_copy(
                        bufs[(c + 1) % 2],
                        out_hbm.at[pl.ds(base + (c - 1) * ch, ch)],
                        wsem[(c + 1) % 2],
                    ).wait()
                gather_start(c + 1)
        pltpu.make_async_copy(
            bufs[(nch - 1) % 2],
            out_hbm.at[pl.ds(base + (nch - 1) * ch, ch)],
            wsem[(nch - 1) % 2],
        ).wait()
        if nch >= 2:
            pltpu.make_async_copy(
                bufs[(nch - 2) % 2],
                out_hbm.at[pl.ds(base + (nch - 2) * ch, ch)],
                wsem[(nch - 2) % 2],
            ).wait()

    return k


def _tc_head(bx: int, tx: int, d: int, v: int, g: int):
    nblk = bx // g

    def body(tok_ref, pos_ref, w_ref, b_ref, out_ref):
        w = w_ref[...]
        bias = b_ref[...]
        pos = pos_ref[...]
        for j in range(g):
            h = tok_ref[pl.ds(j * tx, tx), :] + pos
            out_ref[j] = (
                jnp.dot(h, w, preferred_element_type=jnp.float32) + bias
            )

    return pl.pallas_call(
        body,
        grid=(nblk,),
        in_specs=[
            pl.BlockSpec((g * tx, d), lambda i: (i, 0)),
            pl.BlockSpec((tx, d), lambda i: (0, 0)),
            pl.BlockSpec((d, v), lambda i: (0, 0)),
            pl.BlockSpec((1, v), lambda i: (0, 0)),
        ],
        out_specs=pl.BlockSpec((g, tx, v), lambda i: (i, 0, 0)),
        out_shape=jax.ShapeDtypeStruct((bx, tx, v), jnp.float32),
    )


def kernel(x, tok_table, pos_table, W, b):
    bx, tx = x.shape
    vocab, d = tok_table.shape
    n_tot = bx * tx
    idx = x.reshape(n_tot).astype(jnp.int32)
    tok_emb = _sc_gather(n_tot, d, ch=1600)(idx, tok_table)

    return _tc_head(bx, tx, d, vocab, g=32)(
        tok_emb, pos_table, W, b.reshape(1, vocab)
    )
